# Initial kernel scaffold; baseline (speedup 1.0000x reference)
import jax
import jax.numpy as jnp
from jax.experimental import pallas as pl


def kernel(x, edge_index, degrees):
    raise NotImplementedError("write your pallas kernel here")


# speedup vs baseline: 3.5552x; 3.5552x over previous
import functools

import jax
import jax.numpy as jnp
from jax import lax
from jax.experimental import pallas as pl
from jax.experimental.pallas import tpu as pltpu
from jax.experimental.pallas import tpu_sc as plsc

N_NODES = 10000
D = 128
DH = D // 2
N_SC = 2
N_TILES = 16
W = 128
K = 8
SUP = K * W
N_PAD = 10240
ROWS_PER_TILE = N_PAD // N_TILES


def _sc_body(x_hbm, src_hbm, dst_hbm, deg_hbm, out_hbm,
             src_v, dst_v, deg_v, rows_v, acc, sem, *, n_sup):
    c = lax.axis_index("c")
    s = lax.axis_index("s")

    zeros16 = jnp.zeros((16,), jnp.float32)

    def zrow(r, _):
        for f in range(DH // 16):
            rows_v[r, pl.ds(f * 16, 16)] = zeros16
        return 0

    lax.fori_loop(0, ROWS_PER_TILE, zrow, 0)
    row0 = s * ROWS_PER_TILE
    pltpu.sync_copy(rows_v.at[pl.ds(0, ROWS_PER_TILE)],
                    acc.at[pl.ds(row0, ROWS_PER_TILE)])
    plsc.subcore_barrier()

    n_sup_total = n_sup * N_TILES

    def superchunk(g, _):
        base = s * n_sup * K + g * K
        pltpu.sync_copy(src_hbm.at[pl.ds(base, K)], src_v)
        pltpu.sync_copy(dst_hbm.at[pl.ds(c * n_sup_total * K + base, K)], dst_v)
        pltpu.sync_copy(deg_hbm.at[pl.ds(base * W, SUP)], deg_v)
        cps = [
            pltpu.async_copy(x_hbm.at[dst_v.at[j]],
                             rows_v.at[pl.ds(j * W, W)], sem)
            for j in range(K)
        ]
        for cp in cps:
            cp.wait()

        def scale(q, _):
            dchunk = deg_v[pl.ds(q * 16, 16)]
            for i in range(16):
                e = q * 16 + i
                dv = jnp.full((16,), dchunk[i], jnp.float32)
                for f in range(DH // 16):
                    sl = pl.ds(f * 16, 16)
                    rows_v[e, sl] = rows_v[e, sl] * dv
            return 0

        lax.fori_loop(0, SUP // 16, scale, 0)

        for j in range(K):
            pltpu.sync_copy(rows_v.at[pl.ds(j * W, W)],
                            acc.at[src_v.at[j]], add=True)
        return 0

    lax.fori_loop(0, n_sup, superchunk, 0)
    plsc.subcore_barrier()
    pltpu.sync_copy(acc.at[pl.ds(row0, ROWS_PER_TILE)],
                    out_hbm.at[pl.ds(c * N_PAD + row0, ROWS_PER_TILE)])


def kernel(x, edge_index, degrees):
    n_edges = edge_index.shape[1]
    per_tile = -(-n_edges // (N_TILES * SUP)) * SUP
    e_pad = per_tile * N_TILES
    n_sup = per_tile // SUP
    pad = e_pad - n_edges

    src = edge_index[0]
    dst = edge_index[1]
    deg = degrees
    if pad:
        fill = (jnp.arange(pad, dtype=jnp.int32) * 16) % N_NODES
        src = jnp.concatenate([src, fill])
        dst = jnp.concatenate([dst, fill])
        deg = jnp.concatenate([deg, jnp.zeros((pad,), jnp.float32)])

    src2 = src.reshape(e_pad // W, W)
    dst2 = jnp.concatenate([dst, dst + N_PAD]).reshape(2 * e_pad // W, W)
    zrows = jnp.zeros((N_PAD - N_NODES, DH), jnp.float32)
    x_cat = jnp.concatenate([x[:, :DH], zrows, x[:, DH:], zrows], axis=0)

    body = functools.partial(_sc_body, n_sup=n_sup)
    out_cat = pl.kernel(
        body,
        out_type=jax.ShapeDtypeStruct((2 * N_PAD, DH), jnp.float32),
        mesh=plsc.VectorSubcoreMesh(core_axis_name="c", subcore_axis_name="s",
                                    num_cores=N_SC, num_subcores=N_TILES),
        scratch_types=[
            pltpu.VMEM((K, W), jnp.int32),
            pltpu.VMEM((K, W), jnp.int32),
            pltpu.VMEM((SUP,), jnp.float32),
            pltpu.VMEM((SUP, DH), jnp.float32),
            pltpu.VMEM_SHARED((N_PAD, DH), jnp.float32),
            pltpu.SemaphoreType.DMA,
        ],
        compiler_params=pltpu.CompilerParams(use_tc_tiling_on_sc=False),
    )(x_cat, src2, dst2, deg)

    return jnp.concatenate(
        [out_cat[:N_NODES], out_cat[N_PAD:N_PAD + N_NODES]], axis=1)
